# Initial kernel scaffold; baseline (speedup 1.0000x reference)
import jax
import jax.numpy as jnp
from jax.experimental import pallas as pl


def kernel(token_ids, weight):
    raise NotImplementedError("write your pallas kernel here")


# speedup vs baseline: 4.5426x; 4.5426x over previous
import functools

import jax
import jax.numpy as jnp
from jax import lax
from jax.experimental import pallas as pl
from jax.experimental.pallas import tpu as pltpu
from jax.experimental.pallas import tpu_sc as plsc

CHUNK = 128
NBUF = 2


@functools.cache
def _make_lookup(V, D, B):
    info = plsc.get_sparse_core_info()
    NW = info.num_cores * info.num_subcores
    NC = info.num_cores
    assert B % (NW * CHUNK) == 0
    b_per_w = B // NW
    n_chunks = b_per_w // CHUNK
    assert n_chunks % NBUF == 0

    mesh = plsc.VectorSubcoreMesh(core_axis_name="c", subcore_axis_name="s")

    @functools.partial(
        pl.kernel,
        mesh=mesh,
        compiler_params=pltpu.CompilerParams(use_tc_tiling_on_sc=False),
        out_type=jax.ShapeDtypeStruct((B, D), jnp.float32),
        scratch_types=[
            pltpu.VMEM((n_chunks, CHUNK), jnp.int32),
            pltpu.VMEM((NBUF, CHUNK, D), jnp.float32),
            pltpu.SemaphoreType.DMA,
            pltpu.SemaphoreType.DMA,
        ],
    )
    def lookup(idx_hbm, table_hbm, out_hbm, idx_v, rows_v, sem0, sem1):
        sems = (sem0, sem1)
        wid = lax.axis_index("s") * NC + lax.axis_index("c")
        base = wid * b_per_w
        pltpu.sync_copy(idx_hbm.at[wid], idx_v)
        for b in range(NBUF):
            pltpu.async_copy(table_hbm.at[idx_v.at[b]], rows_v.at[b], sems[b])

        def body(g, carry):
            for b in range(NBUF):
                j = g * NBUF + b
                pltpu.make_async_copy(
                    table_hbm.at[idx_v.at[j]], rows_v.at[b], sems[b]
                ).wait()
                pltpu.sync_copy(
                    rows_v.at[b], out_hbm.at[pl.ds(base + j * CHUNK, CHUNK)]
                )
                jn = j + NBUF

                @pl.when(jn < n_chunks)
                def _():
                    pltpu.async_copy(
                        table_hbm.at[idx_v.at[jn]], rows_v.at[b], sems[b]
                    )
            return carry

        lax.fori_loop(0, n_chunks // NBUF, body, 0)

    return lookup


def kernel(token_ids, weight):
    B1, B2 = token_ids.shape
    V, D = weight.shape
    B = B1 * B2
    info = plsc.get_sparse_core_info()
    NW = info.num_cores * info.num_subcores
    idx = token_ids.astype(jnp.int32).reshape(NW, (B // NW) // CHUNK, CHUNK)
    out = _make_lookup(V, D, B)(idx, weight)
    return out.reshape(B1, B2, D)


# speedup vs baseline: 4.5607x; 1.0040x over previous
import functools

import jax
import jax.numpy as jnp
from jax import lax
from jax.experimental import pallas as pl
from jax.experimental.pallas import tpu as pltpu
from jax.experimental.pallas import tpu_sc as plsc

NBUF = 4


@functools.cache
def _make_lookup(V, D, B1, B2):
    info = plsc.get_sparse_core_info()
    NC = info.num_cores
    NW = NC * info.num_subcores
    assert B1 % (NW * NBUF) == 0
    rows_per_w = B1 // NW

    mesh = plsc.VectorSubcoreMesh(core_axis_name="c", subcore_axis_name="s")

    @functools.partial(
        pl.kernel,
        mesh=mesh,
        compiler_params=pltpu.CompilerParams(use_tc_tiling_on_sc=False),
        out_type=jax.ShapeDtypeStruct((B1, B2, D), jnp.float32),
        scratch_types=[
            pltpu.VMEM((rows_per_w, B2), jnp.int32),
            pltpu.VMEM((NBUF, B2, D), jnp.float32),
            pltpu.SemaphoreType.DMA,
            pltpu.SemaphoreType.DMA,
            pltpu.SemaphoreType.DMA,
            pltpu.SemaphoreType.DMA,
        ],
    )
    def lookup(tok_hbm, table_hbm, out_hbm, idx_v, rows_v, *sems):
        wid = lax.axis_index("s") * NC + lax.axis_index("c")
        base = wid * rows_per_w
        pltpu.sync_copy(tok_hbm.at[pl.ds(base, rows_per_w)], idx_v)
        for b in range(NBUF):
            pltpu.async_copy(table_hbm.at[idx_v.at[b]], rows_v.at[b], sems[b])

        def body(g, carry):
            for b in range(NBUF):
                j = g * NBUF + b
                pltpu.make_async_copy(
                    table_hbm.at[idx_v.at[0]], rows_v.at[b], sems[b]
                ).wait()
                pltpu.sync_copy(rows_v.at[b], out_hbm.at[base + j])
                jn = j + NBUF

                @pl.when(jn < rows_per_w)
                def _():
                    pltpu.async_copy(
                        table_hbm.at[idx_v.at[jn]], rows_v.at[b], sems[b]
                    )
            return carry

        lax.fori_loop(0, rows_per_w // NBUF, body, 0)

    return lookup


def kernel(token_ids, weight):
    B1, B2 = token_ids.shape
    V, D = weight.shape
    return _make_lookup(V, D, B1, B2)(token_ids.astype(jnp.int32), weight)


# speedup vs baseline: 5.3564x; 1.1745x over previous
import functools

import jax
import jax.numpy as jnp
from jax import lax
from jax.experimental import pallas as pl
from jax.experimental.pallas import tpu as pltpu
from jax.experimental.pallas import tpu_sc as plsc


@functools.cache
def _make_lookup(V, D, B1, B2):
    info = plsc.get_sparse_core_info()
    NC = info.num_cores
    L = info.num_lanes
    NW = NC * info.num_subcores
    planes_per_w = D // NW
    assert D % NW == 0 and B1 % L == 0
    n_grp = B1 // L

    mesh = plsc.VectorSubcoreMesh(core_axis_name="c", subcore_axis_name="s")

    @functools.partial(
        pl.kernel,
        mesh=mesh,
        compiler_params=pltpu.CompilerParams(
            use_tc_tiling_on_sc=True, needs_layout_passes=False
        ),
        out_type=jax.ShapeDtypeStruct((B2, D, B1), jnp.float32),
        scratch_types=[
            pltpu.VMEM((V,), jnp.float32),
            pltpu.VMEM((B1,), jnp.int32),
            pltpu.VMEM((B1,), jnp.int32),
            pltpu.VMEM((B1,), jnp.float32),
            pltpu.VMEM((B1,), jnp.float32),
            pltpu.SemaphoreType.DMA,
            pltpu.SemaphoreType.DMA,
            pltpu.SemaphoreType.DMA,
            pltpu.SemaphoreType.DMA,
        ],
    )
    def lookup(tok_hbm, wt_hbm, out_hbm, plane_v, tok0_v, tok1_v,
               row0_v, row1_v, tsem0, tsem1, osem0, osem1):
        toks = (tok0_v, tok1_v)
        rows = (row0_v, row1_v)
        tsems = (tsem0, tsem1)
        osems = (osem0, osem1)
        wid = lax.axis_index("s") * NC + lax.axis_index("c")

        def gather_row(tok_b, row_b):
            def grp(g, carry):
                sl = pl.ds(g * L, L)
                row_b[sl] = plsc.load_gather(plane_v, [tok_b[sl]])
                return carry
            lax.fori_loop(0, n_grp, grp, 0)

        for dd in range(planes_per_w):
            d = wid * planes_per_w + dd
            pltpu.sync_copy(wt_hbm.at[d], plane_v)
            pltpu.async_copy(tok_hbm.at[0], toks[0], tsems[0])
            pltpu.async_copy(tok_hbm.at[1], toks[1], tsems[1])

            def step(g, carry):
                for b in range(2):
                    s = g * 2 + b
                    pltpu.make_async_copy(
                        tok_hbm.at[0], toks[b], tsems[b]
                    ).wait()
                    gather_row(toks[b], rows[b])
                    sn = s + 2

                    @pl.when(sn < B2)
                    def _():
                        pltpu.async_copy(
                            tok_hbm.at[sn], toks[b], tsems[b]
                        )
                    pltpu.async_copy(
                        rows[b], out_hbm.at[s, d], osems[b]
                    )
                    pltpu.make_async_copy(
                        rows[b], out_hbm.at[s, d], osems[b]
                    ).wait()
                return carry

            lax.fori_loop(0, B2 // 2, step, 0)

    return lookup


def kernel(token_ids, weight):
    B1, B2 = token_ids.shape
    V, D = weight.shape
    out_t = _make_lookup(V, D, B1, B2)(
        token_ids.astype(jnp.int32).T, weight.T
    )
    return out_t.transpose(2, 0, 1)


# speedup vs baseline: 7.5155x; 1.4031x over previous
import functools

import jax
import jax.numpy as jnp
from jax import lax
from jax.experimental import pallas as pl
from jax.experimental.pallas import tpu as pltpu
from jax.experimental.pallas import tpu_sc as plsc


@functools.cache
def _make_lookup(V, D, B1, B2):
    info = plsc.get_sparse_core_info()
    NC = info.num_cores
    L = info.num_lanes
    NW = NC * info.num_subcores
    planes_per_w = D // NW
    assert D % NW == 0 and B1 % L == 0
    n_grp = B1 // L

    mesh = plsc.VectorSubcoreMesh(core_axis_name="c", subcore_axis_name="s")

    @functools.partial(
        pl.kernel,
        mesh=mesh,
        compiler_params=pltpu.CompilerParams(
            use_tc_tiling_on_sc=True, needs_layout_passes=False
        ),
        out_type=jax.ShapeDtypeStruct((B2, D, B1), jnp.float32),
        scratch_types=[
            pltpu.VMEM((V,), jnp.float32),
            pltpu.VMEM((B1,), jnp.int32),
            pltpu.VMEM((B1,), jnp.int32),
            pltpu.VMEM((B1,), jnp.float32),
            pltpu.VMEM((B1,), jnp.float32),
            pltpu.SemaphoreType.DMA,
            pltpu.SemaphoreType.DMA,
            pltpu.SemaphoreType.DMA,
            pltpu.SemaphoreType.DMA,
        ],
    )
    def lookup(tok_hbm, wt_hbm, out_hbm, plane_v, tok0_v, tok1_v,
               row0_v, row1_v, tsem0, tsem1, osem0, osem1):
        toks = (tok0_v, tok1_v)
        rows = (row0_v, row1_v)
        tsems = (tsem0, tsem1)
        osems = (osem0, osem1)
        wid = lax.axis_index("s") * NC + lax.axis_index("c")

        UNROLL = 8

        def gather_row(tok_b, row_b):
            def grp(g, carry):
                for u in range(UNROLL):
                    sl = pl.ds((g * UNROLL + u) * L, L)
                    row_b[sl] = plsc.load_gather(plane_v, [tok_b[sl]])
                return carry
            lax.fori_loop(0, n_grp // UNROLL, grp, 0)

        for dd in range(planes_per_w):
            d = wid * planes_per_w + dd
            pltpu.sync_copy(wt_hbm.at[d], plane_v)
            pltpu.async_copy(tok_hbm.at[0], toks[0], tsems[0])
            pltpu.async_copy(tok_hbm.at[1], toks[1], tsems[1])

            def step(g, carry):
                for b in range(2):
                    s = g * 2 + b
                    pltpu.make_async_copy(
                        tok_hbm.at[0], toks[b], tsems[b]
                    ).wait()

                    @pl.when(s >= 2)
                    def _():
                        pltpu.make_async_copy(
                            rows[b], out_hbm.at[0, 0], osems[b]
                        ).wait()

                    gather_row(toks[b], rows[b])
                    sn = s + 2

                    @pl.when(sn < B2)
                    def _():
                        pltpu.async_copy(
                            tok_hbm.at[sn], toks[b], tsems[b]
                        )
                    pltpu.async_copy(
                        rows[b], out_hbm.at[s, d], osems[b]
                    )
                return carry

            lax.fori_loop(0, B2 // 2, step, 0)
            for b in range(2):
                pltpu.make_async_copy(
                    rows[b], out_hbm.at[0, 0], osems[b]
                ).wait()

    return lookup


def kernel(token_ids, weight):
    B1, B2 = token_ids.shape
    V, D = weight.shape
    out_t = _make_lookup(V, D, B1, B2)(
        token_ids.astype(jnp.int32).T, weight.T
    )
    return out_t.transpose(2, 0, 1)


# speedup vs baseline: 7.5635x; 1.0064x over previous
import functools

import jax
import jax.numpy as jnp
from jax import lax
from jax.experimental import pallas as pl
from jax.experimental.pallas import tpu as pltpu
from jax.experimental.pallas import tpu_sc as plsc


@functools.cache
def _make_lookup(V, D, B1, B2):
    info = plsc.get_sparse_core_info()
    NC = info.num_cores
    L = info.num_lanes
    NW = NC * info.num_subcores
    planes_per_w = D // NW
    assert D % NW == 0 and B1 % L == 0
    n_grp = B1 // L

    mesh = plsc.VectorSubcoreMesh(core_axis_name="c", subcore_axis_name="s")

    @functools.partial(
        pl.kernel,
        mesh=mesh,
        compiler_params=pltpu.CompilerParams(
            use_tc_tiling_on_sc=True, needs_layout_passes=False
        ),
        out_type=jax.ShapeDtypeStruct((B2, D, B1), jnp.float32),
        scratch_types=[
            pltpu.VMEM((V,), jnp.float32),
            pltpu.VMEM((B1,), jnp.int32),
            pltpu.VMEM((B1,), jnp.int32),
            pltpu.VMEM((B1,), jnp.float32),
            pltpu.VMEM((B1,), jnp.float32),
            pltpu.SemaphoreType.DMA,
            pltpu.SemaphoreType.DMA,
            pltpu.SemaphoreType.DMA,
            pltpu.SemaphoreType.DMA,
        ],
    )
    def lookup(tok_hbm, wt_hbm, out_hbm, plane_v, tok0_v, tok1_v,
               row0_v, row1_v, tsem0, tsem1, osem0, osem1):
        toks = (tok0_v, tok1_v)
        rows = (row0_v, row1_v)
        tsems = (tsem0, tsem1)
        osems = (osem0, osem1)
        wid = lax.axis_index("s") * NC + lax.axis_index("c")

        UNROLL = 16

        def gather_row(tok_b, row_b):
            def grp(g, carry):
                for u in range(UNROLL):
                    sl = pl.ds((g * UNROLL + u) * L, L)
                    row_b[sl] = plsc.load_gather(plane_v, [tok_b[sl]])
                return carry
            lax.fori_loop(0, n_grp // UNROLL, grp, 0)

        for dd in range(planes_per_w):
            d = wid * planes_per_w + dd
            pltpu.sync_copy(wt_hbm.at[d], plane_v)
            pltpu.async_copy(tok_hbm.at[0], toks[0], tsems[0])
            pltpu.async_copy(tok_hbm.at[1], toks[1], tsems[1])

            def step(g, carry):
                for b in range(2):
                    s = g * 2 + b
                    pltpu.make_async_copy(
                        tok_hbm.at[0], toks[b], tsems[b]
                    ).wait()

                    @pl.when(s >= 2)
                    def _():
                        pltpu.make_async_copy(
                            rows[b], out_hbm.at[0, 0], osems[b]
                        ).wait()

                    gather_row(toks[b], rows[b])
                    sn = s + 2

                    @pl.when(sn < B2)
                    def _():
                        pltpu.async_copy(
                            tok_hbm.at[sn], toks[b], tsems[b]
                        )
                    pltpu.async_copy(
                        rows[b], out_hbm.at[s, d], osems[b]
                    )
                return carry

            lax.fori_loop(0, B2 // 2, step, 0)
            for b in range(2):
                pltpu.make_async_copy(
                    rows[b], out_hbm.at[0, 0], osems[b]
                ).wait()

    return lookup


def kernel(token_ids, weight):
    B1, B2 = token_ids.shape
    V, D = weight.shape
    out_t = _make_lookup(V, D, B1, B2)(
        token_ids.astype(jnp.int32).T, weight.T
    )
    return out_t.transpose(2, 0, 1)


# speedup vs baseline: 7.7338x; 1.0225x over previous
import functools

import jax
import jax.numpy as jnp
from jax import lax
from jax.experimental import pallas as pl
from jax.experimental.pallas import tpu as pltpu
from jax.experimental.pallas import tpu_sc as plsc


@functools.cache
def _make_lookup(V, D, B1, B2):
    info = plsc.get_sparse_core_info()
    NC = info.num_cores
    L = info.num_lanes
    NW = NC * info.num_subcores
    planes_per_w = D // NW
    assert D % NW == 0 and B1 % L == 0
    n_grp = B1 // L

    mesh = plsc.VectorSubcoreMesh(core_axis_name="c", subcore_axis_name="s")

    @functools.partial(
        pl.kernel,
        mesh=mesh,
        compiler_params=pltpu.CompilerParams(
            use_tc_tiling_on_sc=True, needs_layout_passes=False
        ),
        out_type=jax.ShapeDtypeStruct((B2, D, B1), jnp.float32),
        scratch_types=[
            pltpu.VMEM((V,), jnp.float32),
            pltpu.VMEM((B1,), jnp.int32),
            pltpu.VMEM((B1,), jnp.int32),
            pltpu.VMEM((B1,), jnp.int32),
            pltpu.VMEM((B1,), jnp.float32),
            pltpu.VMEM((B1,), jnp.float32),
            pltpu.VMEM((B1,), jnp.float32),
            pltpu.SemaphoreType.DMA,
            pltpu.SemaphoreType.DMA,
            pltpu.SemaphoreType.DMA,
            pltpu.SemaphoreType.DMA,
            pltpu.SemaphoreType.DMA,
            pltpu.SemaphoreType.DMA,
        ],
    )
    def lookup(tok_hbm, wt_hbm, out_hbm, plane_v, tok0_v, tok1_v, tok2_v,
               row0_v, row1_v, row2_v, tsem0, tsem1, tsem2,
               osem0, osem1, osem2):
        toks = (tok0_v, tok1_v, tok2_v)
        rows = (row0_v, row1_v, row2_v)
        tsems = (tsem0, tsem1, tsem2)
        osems = (osem0, osem1, osem2)
        R = 3
        wid = lax.axis_index("s") * NC + lax.axis_index("c")

        UNROLL = 16

        def gather_row(tok_b, row_b):
            def grp(g, carry):
                for u in range(UNROLL):
                    sl = pl.ds((g * UNROLL + u) * L, L)
                    row_b[sl] = plsc.load_gather(plane_v, [tok_b[sl]])
                return carry
            lax.fori_loop(0, n_grp // UNROLL, grp, 0)

        for dd in range(planes_per_w):
            d = wid * planes_per_w + dd
            pltpu.sync_copy(wt_hbm.at[d], plane_v)
            for b in range(R):
                pltpu.async_copy(tok_hbm.at[b], toks[b], tsems[b])

            def body(s, b):
                pltpu.make_async_copy(
                    tok_hbm.at[0], toks[b], tsems[b]
                ).wait()

                @pl.when(s >= R)
                def _():
                    pltpu.make_async_copy(
                        rows[b], out_hbm.at[0, 0], osems[b]
                    ).wait()

                gather_row(toks[b], rows[b])
                sn = s + R

                @pl.when(sn < B2)
                def _():
                    pltpu.async_copy(tok_hbm.at[sn], toks[b], tsems[b])

                pltpu.async_copy(rows[b], out_hbm.at[s, d], osems[b])

            def step(g, carry):
                for k in range(R):
                    body(g * R + k, k)
                return carry

            lax.fori_loop(0, B2 // R, step, 0)
            for s in range(B2 - B2 % R, B2):
                body(jnp.int32(s), s % R)
            for b in range(R):
                pltpu.make_async_copy(
                    rows[b], out_hbm.at[0, 0], osems[b]
                ).wait()

    return lookup


def kernel(token_ids, weight):
    B1, B2 = token_ids.shape
    V, D = weight.shape
    out_t = _make_lookup(V, D, B1, B2)(
        token_ids.astype(jnp.int32).T, weight.T
    )
    return out_t.transpose(2, 0, 1)
